# Initial kernel scaffold; baseline (speedup 1.0000x reference)
import jax
import jax.numpy as jnp
from jax.experimental import pallas as pl


def kernel(s, V, edge_index, edge_s, edge_V, params):
    raise NotImplementedError("write your pallas kernel here")


# speedup vs baseline: 1.5676x; 1.5676x over previous
import functools

import jax
import jax.numpy as jnp
from jax import lax
from jax.experimental import pallas as pl
from jax.experimental.pallas import tpu as pltpu
from jax.experimental.pallas import tpu_sc as plsc

NS = 128
NV = 16
ES = 16
EV = 1
EPS = 1e-4
LN_EPS = 1e-5
TW = NS + 3 * NV

NW = 32
CH = 80
EB = 512
NB = 1000

def _mesh():
    return plsc.VectorSubcoreMesh(core_axis_name="c", subcore_axis_name="s")


def _dot(a, b):
    return lax.dot_general(a, b, (((1,), (0,)), ((), ())),
                           preferred_element_type=jnp.float32)


def _norm3(v0, v1, v2):
    return jnp.maximum(jnp.sqrt(v0 * v0 + v1 * v1 + v2 * v2), EPS)


def _gvp16(s, V, H, M, Wa, Wb, b, relu, sig):
    Vh = [_dot(V[c], H) for c in range(3)]
    sh = _norm3(*Vh)
    sm = _dot(s, Wa) + _dot(sh, Wb) + b
    sd = jnp.maximum(sm, 0.0) if relu else sm
    Vmu = [_dot(Vh[c], M) for c in range(3)]
    vmu = _norm3(*Vmu)
    g = jax.nn.sigmoid(vmu) if sig else vmu
    return sd, [g * Vmu[c] for c in range(3)]


def _ln(x, g, b):
    mu = jnp.mean(x, axis=1, keepdims=True)
    d = x - mu
    var = jnp.mean(d * d, axis=1, keepdims=True)
    return d * lax.rsqrt(var + LN_EPS) * g + b


def _vnorm(V):
    ss = sum(jnp.sum(v * v, axis=1, keepdims=True) for v in V)
    n = jnp.maximum(jnp.sqrt(ss) * 0.25, jnp.sqrt(LN_EPS))
    return [v / n for v in V]


def _edge_body(gd_ref, gs_ref, es_ref, ev_ref,
               A1_ref, A2_ref, a3_ref, M1_ref,
               Wmd_ref, Wms_ref, Wme_ref, Wmh_ref, b1_ref,
               H2_ref, M2_ref, Wa2_ref, Wb2_ref, b2_ref,
               H3_ref, M3_ref, Wa3_ref, Wb3_ref, b3_ref,
               Ha_ref, wa_ref, wb_ref, ba_ref,
               msg_ref):
    gd = gd_ref[...]
    gs = gs_ref[...]
    sd = gd[:, :NS]
    ss = gs[:, :NS]
    es = es_ref[...]
    ev = ev_ref[...]
    Vd = [gd[:, NS + 16 * c:NS + 16 * (c + 1)] for c in range(3)]
    Vs = [gs[:, NS + 16 * c:NS + 16 * (c + 1)] for c in range(3)]

    A1 = A1_ref[...]
    A2 = A2_ref[...]
    a3 = a3_ref[...]
    Vh = [_dot(Vd[c], A1) + _dot(Vs[c], A2) + ev[:, c:c + 1] * a3
          for c in range(3)]
    sh = _norm3(*Vh)
    sm = (_dot(sd, Wmd_ref[...]) + _dot(ss, Wms_ref[...]) +
          _dot(es, Wme_ref[...]) + _dot(sh, Wmh_ref[...]) + b1_ref[...])
    ms = jnp.maximum(sm, 0.0)
    M1 = M1_ref[...]
    Vmu = [_dot(Vh[c], M1) for c in range(3)]
    vmu = _norm3(*Vmu)
    g = jax.nn.sigmoid(vmu)
    mV = [g * Vmu[c] for c in range(3)]

    ms, mV = _gvp16(ms, mV, H2_ref[...], M2_ref[...], Wa2_ref[...],
                    Wb2_ref[...], b2_ref[...], relu=True, sig=True)
    ms, mV = _gvp16(ms, mV, H3_ref[...], M3_ref[...], Wa3_ref[...],
                    Wb3_ref[...], b3_ref[...], relu=False, sig=False)

    Ha = Ha_ref[...]
    Vha = [_dot(mV[c], Ha) for c in range(3)]
    sha = _norm3(*Vha)
    logit = (jnp.sum(ms * wa_ref[...], axis=1, keepdims=True) +
             jnp.sum(sha * wb_ref[...], axis=1, keepdims=True) + ba_ref[0, 0])
    att = jax.nn.sigmoid(logit)

    msg_ref[:, :NS] = att * ms
    for c in range(3):
        msg_ref[:, NS + 16 * c:NS + 16 * (c + 1)] = att * mV[c]


def _edge_weight_shapes():
    return [
        (16, 33), (16, 33), (1, 33), (33, 16),
        (NS, NS), (NS, NS), (ES, NS), (33, NS), (1, NS),
        (16, 16), (16, 16), (NS, NS), (16, NS), (1, NS),
        (16, 16), (16, 16), (NS, NS), (16, NS), (1, NS),
        (16, 16), (1, NS), (1, 16), (1, 1),
    ]


def _full_spec(shape):
    return pl.BlockSpec(shape, lambda i: (0, 0))


def _edge_in_specs():
    data = [
        pl.BlockSpec((EB, TW), lambda i: (i, 0)),
        pl.BlockSpec((EB, TW), lambda i: (i, 0)),
        pl.BlockSpec((EB, ES), lambda i: (i, 0)),
        pl.BlockSpec((EB, 3), lambda i: (i, 0)),
    ]
    return data + [_full_spec(s) for s in _edge_weight_shapes()]


def _node_body(s_ref, vt_ref, a0_ref, a1_ref,
               Hf1_ref, Mf1_ref, Waf1_ref, Wbf1_ref, bf1_ref,
               Hf2_ref, Mf2_ref, Waf2_ref, Wbf2_ref, bf2_ref,
               g1_ref, be1_ref, g2_ref, be2_ref,
               s2_ref, v2_ref):
    agg = a0_ref[...] + a1_ref[...]
    s1 = _ln(s_ref[...] + agg[:, :NS], g1_ref[...], be1_ref[...])
    V1 = _vnorm([vt_ref[:, 16 * c:16 * (c + 1)] +
                 agg[:, NS + 16 * c:NS + 16 * (c + 1)] for c in range(3)])
    fs, fV = _gvp16(s1, V1, Hf1_ref[...], Mf1_ref[...], Waf1_ref[...],
                    Wbf1_ref[...], bf1_ref[...], relu=True, sig=True)
    fs, fV = _gvp16(fs, fV, Hf2_ref[...], Mf2_ref[...], Waf2_ref[...],
                    Wbf2_ref[...], bf2_ref[...], relu=False, sig=False)
    s2_ref[...] = _ln(s1 + fs, g2_ref[...], be2_ref[...])
    V2 = _vnorm([V1[c] + fV[c] for c in range(3)])
    for c in range(3):
        v2_ref[:, 16 * c:16 * (c + 1)] = V2[c]


def _node_weight_shapes():
    return [
        (16, 16), (16, 16), (NS, NS), (16, NS), (1, NS),
        (16, 16), (16, 16), (NS, NS), (16, NS), (1, NS),
        (1, NS), (1, NS), (1, NS), (1, NS),
    ]


def _node_in_specs():
    data = [
        pl.BlockSpec((NB, NS), lambda i: (i, 0)),
        pl.BlockSpec((NB, 3 * NV), lambda i: (i, 0)),
        pl.BlockSpec((NB, TW), lambda i: (i, 0)),
        pl.BlockSpec((NB, TW), lambda i: (i, 0)),
    ]
    return data + [_full_spec(s) for s in _node_weight_shapes()]


def _split_gvp128(p):
    Wh, Wmu, Wm, bm = p
    return (Wh.T, Wmu.T, Wm[:, :NS].T, Wm[:, NS:].T, bm.reshape(1, NS))


def _edge_weights(params):
    Wh1, Wmu1, Wm1, bm1 = params['e1']
    e1 = (Wh1[:, :16].T, Wh1[:, 16:32].T, Wh1[:, 32:33].T, Wmu1.T,
          Wm1[:, :NS].T, Wm1[:, NS:2 * NS].T, Wm1[:, 2 * NS:2 * NS + ES].T,
          Wm1[:, 2 * NS + ES:].T, bm1.reshape(1, NS))
    e2 = _split_gvp128(params['e2'])
    e3 = _split_gvp128(params['e3'])
    Wha, _, Wma, bma = params['att']
    att = (Wha.T, Wma[:1, :NS], Wma[:1, NS:], bma.reshape(1, 1))
    return e1 + e2 + e3 + att


def _node_weights(params):
    f1 = _split_gvp128(params['f1'])
    f2 = _split_gvp128(params['f2'])
    ln = (params['ln1_g'].reshape(1, NS), params['ln1_b'].reshape(1, NS),
          params['ln2_g'].reshape(1, NS), params['ln2_b'].reshape(1, NS))
    return f1 + f2 + ln


def _sc_gather(T, dst, src):
    E = dst.shape[0]
    epw = E // NW
    nch = epw // CH

    @functools.partial(
        pl.kernel,
        out_type=(jax.ShapeDtypeStruct((E, TW), jnp.float32),
                  jax.ShapeDtypeStruct((E, TW), jnp.float32)),
        mesh=_mesh(),
        compiler_params=pltpu.CompilerParams(use_tc_tiling_on_sc=False),
        scratch_types=[
            pltpu.VMEM((CH,), jnp.int32), pltpu.VMEM((CH,), jnp.int32),
            pltpu.VMEM((CH, TW), jnp.float32), pltpu.VMEM((CH, TW), jnp.float32),
            pltpu.SemaphoreType.DMA, pltpu.SemaphoreType.DMA,
        ],
    )
    def k(t_hbm, dst_hbm, src_hbm, od_hbm, os_hbm,
          idx_d, idx_s, buf_d, buf_s, sem_d, sem_s):
        wid = lax.axis_index("s") * 2 + lax.axis_index("c")
        base = wid * epw

        def body(i, carry):
            off = base + i * CH
            pltpu.sync_copy(dst_hbm.at[pl.ds(off, CH)], idx_d)
            pltpu.sync_copy(src_hbm.at[pl.ds(off, CH)], idx_s)
            cd = pltpu.async_copy(t_hbm.at[idx_d], buf_d, sem_d)
            cs = pltpu.async_copy(t_hbm.at[idx_s], buf_s, sem_s)
            cd.wait()
            cs.wait()
            pltpu.sync_copy(buf_d, od_hbm.at[pl.ds(off, CH)])
            pltpu.sync_copy(buf_s, os_hbm.at[pl.ds(off, CH)])
            return carry

        lax.fori_loop(0, nch, body, 0)

    return k(T, dst, src)


def _sc_scatter(msg, dst, zeros):
    E, W = msg.shape
    Nn = zeros.shape[0]
    epw = E // NW
    nch = epw // CH
    nps = Nn // 16

    @functools.partial(
        pl.kernel,
        out_type=jax.ShapeDtypeStruct((2, Nn, W), jnp.float32),
        mesh=_mesh(),
        compiler_params=pltpu.CompilerParams(use_tc_tiling_on_sc=False),
        scratch_types=[
            pltpu.VMEM((CH,), jnp.int32),
            pltpu.VMEM((CH, W), jnp.float32),
            pltpu.VMEM_SHARED((Nn, W), jnp.float32),
        ],
    )
    def k(msg_hbm, dst_hbm, z_hbm, out_hbm, idx_v, buf, acc):
        cid = lax.axis_index("c")
        sid = lax.axis_index("s")
        wid = sid * 2 + cid
        pltpu.sync_copy(z_hbm.at[pl.ds(sid * nps, nps)],
                        acc.at[pl.ds(sid * nps, nps)])
        plsc.subcore_barrier()
        base = wid * epw

        def body(i, carry):
            off = base + i * CH
            pltpu.sync_copy(dst_hbm.at[pl.ds(off, CH)], idx_v)
            pltpu.sync_copy(msg_hbm.at[pl.ds(off, CH)], buf)
            pltpu.sync_copy(buf, acc.at[idx_v], add=True)
            return carry

        lax.fori_loop(0, nch, body, 0)
        plsc.subcore_barrier()
        pltpu.sync_copy(acc.at[pl.ds(sid * nps, nps)],
                        out_hbm.at[cid, pl.ds(sid * nps, nps)])

    return k(msg, dst, zeros)


def kernel(s, V, edge_index, edge_s, edge_V, params):
    N = s.shape[0]
    E = edge_index.shape[1]
    dst = edge_index[1]
    src = edge_index[0]

    vt = V.transpose(0, 2, 1).reshape(N, 3 * NV)
    T = jnp.concatenate([s, vt], axis=1)
    evt = edge_V.reshape(E, 3)

    gd, gs = _sc_gather(T, dst, src)

    msg = pl.pallas_call(
        _edge_body,
        grid=(E // EB,),
        in_specs=_edge_in_specs(),
        out_specs=pl.BlockSpec((EB, TW), lambda i: (i, 0)),
        out_shape=jax.ShapeDtypeStruct((E, TW), jnp.float32),
    )(gd, gs, edge_s, evt, *_edge_weights(params))

    zeros = jnp.zeros((N, TW), dtype=jnp.float32)
    parts = _sc_scatter(msg, dst, zeros)

    s2, v2 = pl.pallas_call(
        _node_body,
        grid=(N // NB,),
        in_specs=_node_in_specs(),
        out_specs=[pl.BlockSpec((NB, NS), lambda i: (i, 0)),
                   pl.BlockSpec((NB, 3 * NV), lambda i: (i, 0))],
        out_shape=[jax.ShapeDtypeStruct((N, NS), jnp.float32),
                   jax.ShapeDtypeStruct((N, 3 * NV), jnp.float32)],
    )(s, vt, parts[0], parts[1], *_node_weights(params))

    V2 = v2.reshape(N, 3, NV).transpose(0, 2, 1)
    return s2, V2
